# baseline (speedup 1.0000x reference)
import functools

import jax
import jax.numpy as jnp
from jax import lax
from jax.experimental import pallas as pl
from jax.experimental.pallas import tpu as pltpu
from jax.experimental.pallas import tpu_sc as plsc

N = 2048
E = 32768
H = 128
NR = 8
NB = 4
BLK = 256
_GRID = N // BLK


def _enc_post_body(sa_ref, sb_ref, x_ref, w_ref, wl_ref, b_ref, o_ref, *, act):
    acc = jnp.dot(x_ref[...], wl_ref[...], preferred_element_type=jnp.float32)
    acc = acc + b_ref[...]
    for r in range(NR // 2):
        acc = acc + jnp.dot(sa_ref[r], w_ref[r], preferred_element_type=jnp.float32)
        acc = acc + jnp.dot(sb_ref[r], w_ref[NR // 2 + r], preferred_element_type=jnp.float32)
    o_ref[...] = jnp.maximum(acc, 0.0) if act else acc


def _enc_post(S, x, W, wl, b, act):
    Sa, Sb = S
    return pl.pallas_call(
        functools.partial(_enc_post_body, act=act),
        grid=(_GRID,),
        in_specs=[
            pl.BlockSpec((NR // 2, BLK, H), lambda i: (0, i, 0)),
            pl.BlockSpec((NR // 2, BLK, H), lambda i: (0, i, 0)),
            pl.BlockSpec((BLK, H), lambda i: (i, 0)),
            pl.BlockSpec((NR, H, H), lambda i: (0, 0, 0)),
            pl.BlockSpec((H, H), lambda i: (0, 0)),
            pl.BlockSpec((1, H), lambda i: (0, 0)),
        ],
        out_specs=pl.BlockSpec((BLK, H), lambda i: (i, 0)),
        out_shape=jax.ShapeDtypeStruct((N, H), jnp.float32),
    )(Sa.reshape(NR // 2, N, H), Sb.reshape(NR // 2, N, H), x, W, wl,
      b.reshape(1, H))


def _z_body(sa_ref, sb_ref, h1_ref, wm_ref, ws_ref, wlm_ref, wls_ref,
            bm_ref, bs_ref, eps_ref, wcat_ref, z_ref, y_ref):
    zm = jnp.dot(h1_ref[...], wlm_ref[...], preferred_element_type=jnp.float32) + bm_ref[...]
    zs = jnp.dot(h1_ref[...], wls_ref[...], preferred_element_type=jnp.float32) + bs_ref[...]
    for r in range(NR // 2):
        zm = zm + jnp.dot(sa_ref[r], wm_ref[r], preferred_element_type=jnp.float32)
        zm = zm + jnp.dot(sb_ref[r], wm_ref[NR // 2 + r], preferred_element_type=jnp.float32)
        zs = zs + jnp.dot(sa_ref[r], ws_ref[r], preferred_element_type=jnp.float32)
        zs = zs + jnp.dot(sb_ref[r], ws_ref[NR // 2 + r], preferred_element_type=jnp.float32)
    z = zm + zs * eps_ref[...]
    z_ref[...] = z
    y_ref[...] = jnp.dot(z, wcat_ref[...], preferred_element_type=jnp.float32)


def _z_kernel(S, h1, Wm, Ws, wlm, wls, bm, bs, eps, Wcat):
    return pl.pallas_call(
        _z_body,
        grid=(_GRID,),
        in_specs=[
            pl.BlockSpec((NR // 2, BLK, H), lambda i: (0, i, 0)),
            pl.BlockSpec((NR // 2, BLK, H), lambda i: (0, i, 0)),
            pl.BlockSpec((BLK, H), lambda i: (i, 0)),
            pl.BlockSpec((NR, H, H), lambda i: (0, 0, 0)),
            pl.BlockSpec((NR, H, H), lambda i: (0, 0, 0)),
            pl.BlockSpec((H, H), lambda i: (0, 0)),
            pl.BlockSpec((H, H), lambda i: (0, 0)),
            pl.BlockSpec((1, H), lambda i: (0, 0)),
            pl.BlockSpec((1, H), lambda i: (0, 0)),
            pl.BlockSpec((BLK, H), lambda i: (i, 0)),
            pl.BlockSpec((H, 3 * H), lambda i: (0, 0)),
        ],
        out_specs=[
            pl.BlockSpec((BLK, H), lambda i: (i, 0)),
            pl.BlockSpec((BLK, 3 * H), lambda i: (i, 0)),
        ],
        out_shape=[
            jax.ShapeDtypeStruct((N, H), jnp.float32),
            jax.ShapeDtypeStruct((N, 3 * H), jnp.float32),
        ],
    )(S[0].reshape(NR // 2, N, H), S[1].reshape(NR // 2, N, H), h1, Wm, Ws,
      wlm, wls, bm.reshape(1, H), bs.reshape(1, H), eps, Wcat)


def _ru_body(hb_ref, hf_ref, y_ref, ysum_ref, skip_ref, sc_ref, o_ref):
    A = jax.nn.sigmoid(
        lax.dot_general(hb_ref[...], hf_ref[...], (((1,), (1,)), ((), ())),
                        preferred_element_type=jnp.float32))
    rs = jnp.sum(A, axis=1, keepdims=True) + 1e-8
    Rm = A / rs
    out = (sc_ref[0, 0] * jnp.dot(Rm, y_ref[...], preferred_element_type=jnp.float32)
           + sc_ref[0, 1] * ysum_ref[...] + skip_ref[...])
    o_ref[...] = jnp.maximum(out, 0.0)


def _refine_update(hsrc, y, ysum, skip, scale_shift):
    return pl.pallas_call(
        _ru_body,
        grid=(_GRID,),
        in_specs=[
            pl.BlockSpec((BLK, H), lambda i: (i, 0)),
            pl.BlockSpec((N, H), lambda i: (0, 0)),
            pl.BlockSpec((N, H), lambda i: (0, 0)),
            pl.BlockSpec((1, H), lambda i: (0, 0)),
            pl.BlockSpec((BLK, H), lambda i: (i, 0)),
            pl.BlockSpec(memory_space=pltpu.SMEM),
        ],
        out_specs=pl.BlockSpec((BLK, H), lambda i: (i, 0)),
        out_shape=jax.ShapeDtypeStruct((N, H), jnp.float32),
    )(hsrc, hsrc, y, ysum, skip, scale_shift)


def _rm_body(hb_ref, hf_ref, r_ref, ec_ref):
    gb = jnp.maximum(hb_ref[...], 0.0)
    gf = jnp.maximum(hf_ref[...], 0.0)
    A = jax.nn.sigmoid(
        lax.dot_general(gb, gf, (((1,), (1,)), ((), ())),
                        preferred_element_type=jnp.float32))
    Rm = A / (jnp.sum(A, axis=1, keepdims=True) + 1e-8)
    r_ref[...] = Rm
    ec_ref[...] = jnp.concatenate(
        [Rm[:, :1], Rm[:, N - 1:N], jnp.zeros((BLK, 126), jnp.float32)], axis=1)


def _refine_mat(hsrc):
    return pl.pallas_call(
        _rm_body,
        grid=(_GRID,),
        in_specs=[
            pl.BlockSpec((BLK, H), lambda i: (i, 0)),
            pl.BlockSpec((N, H), lambda i: (0, 0)),
        ],
        out_specs=[
            pl.BlockSpec((BLK, N), lambda i: (i, 0)),
            pl.BlockSpec((BLK, H), lambda i: (i, 0)),
        ],
        out_shape=[
            jax.ShapeDtypeStruct((N, N), jnp.float32),
            jax.ShapeDtypeStruct((N, H), jnp.float32),
        ],
    )(hsrc, hsrc)


def _mv_body(r2_ref, yt_ref, corr_ref, o_ref):
    o_ref[...] = jnp.maximum(
        jnp.dot(r2_ref[...], yt_ref[...], preferred_element_type=jnp.float32)
        + corr_ref[...], 0.0)


def _stencil_mv(R2, yt, corr):
    return pl.pallas_call(
        _mv_body,
        grid=(_GRID,),
        in_specs=[
            pl.BlockSpec((BLK, N), lambda i: (i, 0)),
            pl.BlockSpec((N, H), lambda i: (0, 0)),
            pl.BlockSpec((BLK, H), lambda i: (i, 0)),
        ],
        out_specs=pl.BlockSpec((BLK, H), lambda i: (i, 0)),
        out_shape=jax.ShapeDtypeStruct((N, H), jnp.float32),
    )(R2, yt, corr)


_NC, _NS = 2, 16
_EPT = E // _NS
_GSZ = 128
_NCHUNK = _EPT // _GSZ
_WROW = _EPT // 8
_ACC_ROWS = 2 * N
_RPT = _ACC_ROWS // _NS


def _scale_chunk(rows, wv, i, half):
    def _edge8(g8, c2):
        wrow = i * (_GSZ // 8) + half * (_GSZ // 16) + g8
        for k in range(8):
            wg = wv[wrow, pl.ds(k * 16, 16)]
            g = half * (_GSZ // 2) + g8 * 8 + k
            for f in range(H // 16):
                rows[g, pl.ds(f * 16, 16)] = rows[g, pl.ds(f * 16, 16)] * wg
        return c2
    lax.fori_loop(0, _GSZ // 16, _edge8, 0)


def _sc_scatter_body(x_hbm, src_hbm, sidx_hbm, w_hbm, out_hbm,
                     srcv, sidxv, wv, rows0, rows1, stage, Sacc,
                     gsem0, gsem1, ssem0, ssem1):
    c = lax.axis_index("c")
    s = lax.axis_index("s")
    rows = (rows0, rows1)
    gsem = (gsem0, gsem1)
    ssem = (ssem0, ssem1)

    pltpu.sync_copy(src_hbm.at[s], srcv)
    pltpu.sync_copy(sidx_hbm.at[s], sidxv)
    pltpu.sync_copy(w_hbm.at[c, s], wv)

    def _zrow(i, carry):
        for f in range(H // 16):
            stage[i, pl.ds(f * 16, 16)] = jnp.zeros((16,), jnp.float32)
        return carry
    lax.fori_loop(0, 128, _zrow, 0)

    def _zcp(j, carry):
        pltpu.sync_copy(stage, Sacc.at[pl.ds(s * _RPT + j * 128, 128)])
        return carry
    lax.fori_loop(0, _RPT // 128, _zcp, 0)
    plsc.subcore_barrier()

    pltpu.async_copy(x_hbm.at[srcv.at[0]], rows0, gsem0)

    def _pair(i2, carry):
        for b in range(2):
            ch = i2 * 2 + b
            @pl.when(jnp.logical_and(ch >= 1, ch + 1 < _NCHUNK))
            def _():
                pltpu.make_async_copy(
                    rows[1 - b], Sacc.at[sidxv.at[ch - 1]], ssem[1 - b]).wait()

            @pl.when(ch + 1 < _NCHUNK)
            def _():
                pltpu.async_copy(x_hbm.at[srcv.at[ch + 1]], rows[1 - b], gsem[1 - b])

            pltpu.make_async_copy(x_hbm.at[srcv.at[ch]], rows[b], gsem[b]).wait()
            for half in range(2):
                _scale_chunk(rows[b], wv, ch, half)
            pltpu.async_copy(rows[b], Sacc.at[sidxv.at[ch]], ssem[b])
        return carry
    lax.fori_loop(0, _NCHUNK // 2, _pair, 0)

    pltpu.make_async_copy(rows0, Sacc.at[sidxv.at[_NCHUNK - 2]], ssem0).wait()
    pltpu.make_async_copy(rows1, Sacc.at[sidxv.at[_NCHUNK - 1]], ssem1).wait()
    plsc.subcore_barrier()

    off = s * _RPT
    pltpu.sync_copy(Sacc.at[pl.ds(off, _RPT)],
                    out_hbm.at[pl.ds(c * _ACC_ROWS + off, _RPT)])


_sc_scatter_call = functools.partial(
    pl.kernel,
    mesh=plsc.VectorSubcoreMesh(core_axis_name="c", subcore_axis_name="s"),
    out_type=jax.ShapeDtypeStruct((2 * _ACC_ROWS, H), jnp.float32),
    scratch_types=[
        pltpu.VMEM((_NCHUNK, _GSZ), jnp.int32),
        pltpu.VMEM((_NCHUNK, _GSZ), jnp.int32),
        pltpu.VMEM((_WROW, 128), jnp.float32),
        pltpu.VMEM((_GSZ, H), jnp.float32),
        pltpu.VMEM((_GSZ, H), jnp.float32),
        pltpu.VMEM((128, H), jnp.float32),
        pltpu.VMEM_SHARED((_ACC_ROWS, H), jnp.float32),
        pltpu.SemaphoreType.DMA,
        pltpu.SemaphoreType.DMA,
        pltpu.SemaphoreType.DMA,
        pltpu.SemaphoreType.DMA,
    ],
)

_sc_scatter_fn = _sc_scatter_call(_sc_scatter_body)


def _segment_scatter(x, src, sidx, wp):
    def one(p):
        wexp = jnp.broadcast_to(wp[p].reshape(_NC, _NS, _WROW, 8, 1),
                                (_NC, _NS, _WROW, 8, 16)).reshape(_NC, _NS, _WROW, 128)
        return _sc_scatter_fn(x,
                              src.reshape(_NS, _NCHUNK, _GSZ),
                              sidx.reshape(_NS, _NCHUNK, _GSZ),
                              wexp)
    return one(0), one(1)


def kernel(edge_index, h, r, norm, emb, rc_bases, rc_coef, rc_wloop, rc_bias,
           zm_bases, zm_coef, zm_wloop, zm_bias, zs_bases, zs_coef, zs_wloop,
           zs_bias, W1, W2, W3, conv1_w, conv1_b, conv2_w, conv2_b):
    src = edge_index[0].astype(jnp.int32)
    dst = edge_index[1].astype(jnp.int32)
    ri = r.astype(jnp.int32)
    x0 = emb

    sidx = (ri % 2) * N + dst
    nrm = norm[:, 0]
    rq = ri >> 1
    wp = jnp.stack([jnp.stack([jnp.where(rq == 2 * p + c, nrm, 0.0)
                               for c in range(_NC)]) for p in range(2)])

    Wrc = jnp.einsum('rb,bio->rio', rc_coef, rc_bases)
    Wzm = jnp.einsum('rb,bio->rio', zm_coef, zm_bases)
    Wzs = jnp.einsum('rb,bio->rio', zs_coef, zs_bases)
    Wcat = jnp.concatenate([W1, W2, W3], axis=1)

    eps = jax.random.normal(jax.random.key(1), (N, H), dtype=jnp.float32)

    S1 = _segment_scatter(x0, src, sidx, wp)
    h1 = _enc_post(S1, x0, Wrc, rc_wloop, rc_bias, act=True)
    S2 = _segment_scatter(h1, src, sidx, wp)
    z, y = _z_kernel(S2, h1, Wzm, Wzs, zm_wloop, zs_wloop, zm_bias, zs_bias,
                     eps, Wcat)
    y1, y2, y3 = y[:, :H], y[:, H:2 * H], y[:, 2 * H:]

    one_zero = jnp.array([[1.0, 0.0]], jnp.float32)
    hd1 = _refine_update(z, y1, jnp.zeros((1, H), jnp.float32), x0, one_zero)

    a = jnp.mean(conv1_w[:, 0])
    c = jnp.mean(conv1_b)
    ysum2 = jnp.sum(y2, axis=0, keepdims=True)
    hd2 = _refine_update(hd1, y2, ysum2, h1,
                         jnp.stack([a, c]).reshape(1, 2))

    R2, ec = _refine_mat(hd2)
    u = jnp.mean(jnp.sum(conv2_w, axis=1), axis=0)
    c2 = jnp.mean(conv2_b)
    y3p = jnp.concatenate([y3[1:], jnp.zeros((1, H), jnp.float32)], 0)
    y3m = jnp.concatenate([jnp.zeros((1, H), jnp.float32), y3[:-1]], 0)
    yt = u[1] * y3 + u[0] * y3p + u[2] * y3m
    firstcol, lastcol = ec[:, 0], ec[:, 1]
    prev_last = jnp.concatenate([jnp.zeros((1,), jnp.float32), lastcol[:-1]])
    next_first = jnp.concatenate([firstcol[1:], jnp.zeros((1,), jnp.float32)])
    corr = (u[0] * prev_last[:, None] * y3[0][None, :]
            + u[2] * next_first[:, None] * y3[N - 1][None, :]
            + c2 * jnp.sum(y3, axis=0)[None, :])
    hd3 = _stencil_mv(R2, yt, corr)

    out, _ = _refine_mat(hd3)
    return out


# speedup vs baseline: 1.0067x; 1.0067x over previous
import functools

import jax
import jax.numpy as jnp
from jax import lax
from jax.experimental import pallas as pl
from jax.experimental.pallas import tpu as pltpu
from jax.experimental.pallas import tpu_sc as plsc

N = 2048
E = 32768
H = 128
NR = 8
NB = 4
BLK = 256
_GRID = N // BLK


def _enc_post_body(sa_ref, sb_ref, x_ref, w_ref, wl_ref, b_ref, o_ref, *, act):
    acc = jnp.dot(x_ref[...], wl_ref[...], preferred_element_type=jnp.float32)
    acc = acc + b_ref[...]
    for r in range(NR // 2):
        acc = acc + jnp.dot(sa_ref[r], w_ref[r], preferred_element_type=jnp.float32)
        acc = acc + jnp.dot(sb_ref[r], w_ref[NR // 2 + r], preferred_element_type=jnp.float32)
    o_ref[...] = jnp.maximum(acc, 0.0) if act else acc


def _enc_post(S, x, W, wl, b, act):
    Sa, Sb = S
    return pl.pallas_call(
        functools.partial(_enc_post_body, act=act),
        grid=(_GRID,),
        in_specs=[
            pl.BlockSpec((NR // 2, BLK, H), lambda i: (0, i, 0)),
            pl.BlockSpec((NR // 2, BLK, H), lambda i: (0, i, 0)),
            pl.BlockSpec((BLK, H), lambda i: (i, 0)),
            pl.BlockSpec((NR, H, H), lambda i: (0, 0, 0)),
            pl.BlockSpec((H, H), lambda i: (0, 0)),
            pl.BlockSpec((1, H), lambda i: (0, 0)),
        ],
        out_specs=pl.BlockSpec((BLK, H), lambda i: (i, 0)),
        out_shape=jax.ShapeDtypeStruct((N, H), jnp.float32),
    )(Sa.reshape(NR // 2, N, H), Sb.reshape(NR // 2, N, H), x, W, wl,
      b.reshape(1, H))


def _z_body(sa_ref, sb_ref, h1_ref, wm_ref, ws_ref, wlm_ref, wls_ref,
            bm_ref, bs_ref, eps_ref, wcat_ref, z_ref, y_ref):
    zm = jnp.dot(h1_ref[...], wlm_ref[...], preferred_element_type=jnp.float32) + bm_ref[...]
    zs = jnp.dot(h1_ref[...], wls_ref[...], preferred_element_type=jnp.float32) + bs_ref[...]
    for r in range(NR // 2):
        zm = zm + jnp.dot(sa_ref[r], wm_ref[r], preferred_element_type=jnp.float32)
        zm = zm + jnp.dot(sb_ref[r], wm_ref[NR // 2 + r], preferred_element_type=jnp.float32)
        zs = zs + jnp.dot(sa_ref[r], ws_ref[r], preferred_element_type=jnp.float32)
        zs = zs + jnp.dot(sb_ref[r], ws_ref[NR // 2 + r], preferred_element_type=jnp.float32)
    z = zm + zs * eps_ref[...]
    z_ref[...] = z
    y_ref[...] = jnp.dot(z, wcat_ref[...], preferred_element_type=jnp.float32)


def _z_kernel(S, h1, Wm, Ws, wlm, wls, bm, bs, eps, Wcat):
    return pl.pallas_call(
        _z_body,
        grid=(_GRID,),
        in_specs=[
            pl.BlockSpec((NR // 2, BLK, H), lambda i: (0, i, 0)),
            pl.BlockSpec((NR // 2, BLK, H), lambda i: (0, i, 0)),
            pl.BlockSpec((BLK, H), lambda i: (i, 0)),
            pl.BlockSpec((NR, H, H), lambda i: (0, 0, 0)),
            pl.BlockSpec((NR, H, H), lambda i: (0, 0, 0)),
            pl.BlockSpec((H, H), lambda i: (0, 0)),
            pl.BlockSpec((H, H), lambda i: (0, 0)),
            pl.BlockSpec((1, H), lambda i: (0, 0)),
            pl.BlockSpec((1, H), lambda i: (0, 0)),
            pl.BlockSpec((BLK, H), lambda i: (i, 0)),
            pl.BlockSpec((H, 3 * H), lambda i: (0, 0)),
        ],
        out_specs=[
            pl.BlockSpec((BLK, H), lambda i: (i, 0)),
            pl.BlockSpec((BLK, 3 * H), lambda i: (i, 0)),
        ],
        out_shape=[
            jax.ShapeDtypeStruct((N, H), jnp.float32),
            jax.ShapeDtypeStruct((N, 3 * H), jnp.float32),
        ],
    )(S[0].reshape(NR // 2, N, H), S[1].reshape(NR // 2, N, H), h1, Wm, Ws,
      wlm, wls, bm.reshape(1, H), bs.reshape(1, H), eps, Wcat)


def _ru_body(hb_ref, hf_ref, y_ref, ysum_ref, skip_ref, sc_ref, o_ref):
    A = jax.nn.sigmoid(
        lax.dot_general(hb_ref[...].astype(jnp.bfloat16),
                        hf_ref[...].astype(jnp.bfloat16), (((1,), (1,)), ((), ())),
                        preferred_element_type=jnp.float32))
    rs = jnp.sum(A, axis=1, keepdims=True) + 1e-8
    Rm = A / rs
    out = (sc_ref[0, 0] * jnp.dot(Rm.astype(jnp.bfloat16),
                                  y_ref[...].astype(jnp.bfloat16),
                                  preferred_element_type=jnp.float32)
           + sc_ref[0, 1] * ysum_ref[...] + skip_ref[...])
    o_ref[...] = jnp.maximum(out, 0.0)


def _refine_update(hsrc, y, ysum, skip, scale_shift):
    return pl.pallas_call(
        _ru_body,
        grid=(_GRID,),
        in_specs=[
            pl.BlockSpec((BLK, H), lambda i: (i, 0)),
            pl.BlockSpec((N, H), lambda i: (0, 0)),
            pl.BlockSpec((N, H), lambda i: (0, 0)),
            pl.BlockSpec((1, H), lambda i: (0, 0)),
            pl.BlockSpec((BLK, H), lambda i: (i, 0)),
            pl.BlockSpec(memory_space=pltpu.SMEM),
        ],
        out_specs=pl.BlockSpec((BLK, H), lambda i: (i, 0)),
        out_shape=jax.ShapeDtypeStruct((N, H), jnp.float32),
    )(hsrc, hsrc, y, ysum, skip, scale_shift)


def _rm_body(hb_ref, hf_ref, r_ref, ec_ref):
    gb = jnp.maximum(hb_ref[...], 0.0)
    gf = jnp.maximum(hf_ref[...], 0.0)
    A = jax.nn.sigmoid(
        lax.dot_general(gb.astype(jnp.bfloat16), gf.astype(jnp.bfloat16),
                        (((1,), (1,)), ((), ())),
                        preferred_element_type=jnp.float32))
    Rm = A / (jnp.sum(A, axis=1, keepdims=True) + 1e-8)
    r_ref[...] = Rm
    ec_ref[...] = jnp.concatenate(
        [Rm[:, :1], Rm[:, N - 1:N], jnp.zeros((BLK, 126), jnp.float32)], axis=1)


def _refine_mat(hsrc):
    return pl.pallas_call(
        _rm_body,
        grid=(_GRID,),
        in_specs=[
            pl.BlockSpec((BLK, H), lambda i: (i, 0)),
            pl.BlockSpec((N, H), lambda i: (0, 0)),
        ],
        out_specs=[
            pl.BlockSpec((BLK, N), lambda i: (i, 0)),
            pl.BlockSpec((BLK, H), lambda i: (i, 0)),
        ],
        out_shape=[
            jax.ShapeDtypeStruct((N, N), jnp.float32),
            jax.ShapeDtypeStruct((N, H), jnp.float32),
        ],
    )(hsrc, hsrc)


def _mv_body(r2_ref, yt_ref, corr_ref, o_ref):
    o_ref[...] = jnp.maximum(
        jnp.dot(r2_ref[...].astype(jnp.bfloat16),
                yt_ref[...].astype(jnp.bfloat16),
                preferred_element_type=jnp.float32)
        + corr_ref[...], 0.0)


def _stencil_mv(R2, yt, corr):
    return pl.pallas_call(
        _mv_body,
        grid=(_GRID,),
        in_specs=[
            pl.BlockSpec((BLK, N), lambda i: (i, 0)),
            pl.BlockSpec((N, H), lambda i: (0, 0)),
            pl.BlockSpec((BLK, H), lambda i: (i, 0)),
        ],
        out_specs=pl.BlockSpec((BLK, H), lambda i: (i, 0)),
        out_shape=jax.ShapeDtypeStruct((N, H), jnp.float32),
    )(R2, yt, corr)


_NC, _NS = 2, 16
_EPT = E // _NS
_GSZ = 128
_NCHUNK = _EPT // _GSZ
_WROW = _EPT // 8
_ACC_ROWS = 2 * N
_RPT = _ACC_ROWS // _NS


def _scale_chunk(rows, wv, i, half):
    def _edge8(g8, c2):
        wrow = i * (_GSZ // 8) + half * (_GSZ // 16) + g8
        for k in range(8):
            wg = wv[wrow, pl.ds(k * 16, 16)]
            g = half * (_GSZ // 2) + g8 * 8 + k
            for f in range(H // 16):
                rows[g, pl.ds(f * 16, 16)] = rows[g, pl.ds(f * 16, 16)] * wg
        return c2
    lax.fori_loop(0, _GSZ // 16, _edge8, 0)


def _sc_scatter_body(x_hbm, src_hbm, sidx_hbm, w_hbm, out_hbm,
                     srcv, sidxv, wv, rows0, rows1, stage, Sacc,
                     gsem0, gsem1, ssem0, ssem1):
    c = lax.axis_index("c")
    s = lax.axis_index("s")
    rows = (rows0, rows1)
    gsem = (gsem0, gsem1)
    ssem = (ssem0, ssem1)

    pltpu.sync_copy(src_hbm.at[s], srcv)
    pltpu.sync_copy(sidx_hbm.at[s], sidxv)
    pltpu.sync_copy(w_hbm.at[c, s], wv)

    def _zrow(i, carry):
        for f in range(H // 16):
            stage[i, pl.ds(f * 16, 16)] = jnp.zeros((16,), jnp.float32)
        return carry
    lax.fori_loop(0, 128, _zrow, 0)

    def _zcp(j, carry):
        pltpu.sync_copy(stage, Sacc.at[pl.ds(s * _RPT + j * 128, 128)])
        return carry
    lax.fori_loop(0, _RPT // 128, _zcp, 0)
    plsc.subcore_barrier()

    pltpu.async_copy(x_hbm.at[srcv.at[0]], rows0, gsem0)

    def _pair(i2, carry):
        for b in range(2):
            ch = i2 * 2 + b
            @pl.when(jnp.logical_and(ch >= 1, ch + 1 < _NCHUNK))
            def _():
                pltpu.make_async_copy(
                    rows[1 - b], Sacc.at[sidxv.at[ch - 1]], ssem[1 - b]).wait()

            @pl.when(ch + 1 < _NCHUNK)
            def _():
                pltpu.async_copy(x_hbm.at[srcv.at[ch + 1]], rows[1 - b], gsem[1 - b])

            pltpu.make_async_copy(x_hbm.at[srcv.at[ch]], rows[b], gsem[b]).wait()
            for half in range(2):
                _scale_chunk(rows[b], wv, ch, half)
            pltpu.async_copy(rows[b], Sacc.at[sidxv.at[ch]], ssem[b])
        return carry
    lax.fori_loop(0, _NCHUNK // 2, _pair, 0)

    pltpu.make_async_copy(rows0, Sacc.at[sidxv.at[_NCHUNK - 2]], ssem0).wait()
    pltpu.make_async_copy(rows1, Sacc.at[sidxv.at[_NCHUNK - 1]], ssem1).wait()
    plsc.subcore_barrier()

    off = s * _RPT
    pltpu.sync_copy(Sacc.at[pl.ds(off, _RPT)],
                    out_hbm.at[pl.ds(c * _ACC_ROWS + off, _RPT)])


_sc_scatter_call = functools.partial(
    pl.kernel,
    mesh=plsc.VectorSubcoreMesh(core_axis_name="c", subcore_axis_name="s"),
    out_type=jax.ShapeDtypeStruct((2 * _ACC_ROWS, H), jnp.float32),
    scratch_types=[
        pltpu.VMEM((_NCHUNK, _GSZ), jnp.int32),
        pltpu.VMEM((_NCHUNK, _GSZ), jnp.int32),
        pltpu.VMEM((_WROW, 128), jnp.float32),
        pltpu.VMEM((_GSZ, H), jnp.float32),
        pltpu.VMEM((_GSZ, H), jnp.float32),
        pltpu.VMEM((128, H), jnp.float32),
        pltpu.VMEM_SHARED((_ACC_ROWS, H), jnp.float32),
        pltpu.SemaphoreType.DMA,
        pltpu.SemaphoreType.DMA,
        pltpu.SemaphoreType.DMA,
        pltpu.SemaphoreType.DMA,
    ],
)

_sc_scatter_fn = _sc_scatter_call(_sc_scatter_body)


def _segment_scatter(x, src, sidx, wp):
    def one(p):
        wexp = jnp.broadcast_to(wp[p].reshape(_NC, _NS, _WROW, 8, 1),
                                (_NC, _NS, _WROW, 8, 16)).reshape(_NC, _NS, _WROW, 128)
        return _sc_scatter_fn(x,
                              src.reshape(_NS, _NCHUNK, _GSZ),
                              sidx.reshape(_NS, _NCHUNK, _GSZ),
                              wexp)
    return one(0), one(1)


def kernel(edge_index, h, r, norm, emb, rc_bases, rc_coef, rc_wloop, rc_bias,
           zm_bases, zm_coef, zm_wloop, zm_bias, zs_bases, zs_coef, zs_wloop,
           zs_bias, W1, W2, W3, conv1_w, conv1_b, conv2_w, conv2_b):
    src = edge_index[0].astype(jnp.int32)
    dst = edge_index[1].astype(jnp.int32)
    ri = r.astype(jnp.int32)
    x0 = emb

    sidx = (ri % 2) * N + dst
    nrm = norm[:, 0]
    rq = ri >> 1
    wp = jnp.stack([jnp.stack([jnp.where(rq == 2 * p + c, nrm, 0.0)
                               for c in range(_NC)]) for p in range(2)])

    Wrc = jnp.einsum('rb,bio->rio', rc_coef, rc_bases)
    Wzm = jnp.einsum('rb,bio->rio', zm_coef, zm_bases)
    Wzs = jnp.einsum('rb,bio->rio', zs_coef, zs_bases)
    Wcat = jnp.concatenate([W1, W2, W3], axis=1)

    eps = jax.random.normal(jax.random.key(1), (N, H), dtype=jnp.float32)

    S1 = _segment_scatter(x0, src, sidx, wp)
    h1 = _enc_post(S1, x0, Wrc, rc_wloop, rc_bias, act=True)
    S2 = _segment_scatter(h1, src, sidx, wp)
    z, y = _z_kernel(S2, h1, Wzm, Wzs, zm_wloop, zs_wloop, zm_bias, zs_bias,
                     eps, Wcat)
    y1, y2, y3 = y[:, :H], y[:, H:2 * H], y[:, 2 * H:]

    one_zero = jnp.array([[1.0, 0.0]], jnp.float32)
    hd1 = _refine_update(z, y1, jnp.zeros((1, H), jnp.float32), x0, one_zero)

    a = jnp.mean(conv1_w[:, 0])
    c = jnp.mean(conv1_b)
    ysum2 = jnp.sum(y2, axis=0, keepdims=True)
    hd2 = _refine_update(hd1, y2, ysum2, h1,
                         jnp.stack([a, c]).reshape(1, 2))

    R2, ec = _refine_mat(hd2)
    u = jnp.mean(jnp.sum(conv2_w, axis=1), axis=0)
    c2 = jnp.mean(conv2_b)
    y3p = jnp.concatenate([y3[1:], jnp.zeros((1, H), jnp.float32)], 0)
    y3m = jnp.concatenate([jnp.zeros((1, H), jnp.float32), y3[:-1]], 0)
    yt = u[1] * y3 + u[0] * y3p + u[2] * y3m
    firstcol, lastcol = ec[:, 0], ec[:, 1]
    prev_last = jnp.concatenate([jnp.zeros((1,), jnp.float32), lastcol[:-1]])
    next_first = jnp.concatenate([firstcol[1:], jnp.zeros((1,), jnp.float32)])
    corr = (u[0] * prev_last[:, None] * y3[0][None, :]
            + u[2] * next_first[:, None] * y3[N - 1][None, :]
            + c2 * jnp.sum(y3, axis=0)[None, :])
    hd3 = _stencil_mv(R2, yt, corr)

    out, _ = _refine_mat(hd3)
    return out


# speedup vs baseline: 1.0196x; 1.0128x over previous
import functools

import jax
import jax.numpy as jnp
from jax import lax
from jax.experimental import pallas as pl
from jax.experimental.pallas import tpu as pltpu
from jax.experimental.pallas import tpu_sc as plsc

N = 2048
E = 32768
H = 128
NR = 8
NB = 4
BLK = 256
_GRID = N // BLK


def _enc_post_body(sa_ref, sb_ref, x_ref, w_ref, wl_ref, b_ref, o_ref, *, act):
    acc = jnp.dot(x_ref[...], wl_ref[...], preferred_element_type=jnp.float32)
    acc = acc + b_ref[...]
    for r in range(NR // 2):
        acc = acc + jnp.dot(sa_ref[r], w_ref[r], preferred_element_type=jnp.float32)
        acc = acc + jnp.dot(sb_ref[r], w_ref[NR // 2 + r], preferred_element_type=jnp.float32)
    o_ref[...] = jnp.maximum(acc, 0.0) if act else acc


def _enc_post(S, x, W, wl, b, act):
    Sa, Sb = S
    return pl.pallas_call(
        functools.partial(_enc_post_body, act=act),
        grid=(_GRID,),
        in_specs=[
            pl.BlockSpec((NR // 2, BLK, H), lambda i: (0, i, 0)),
            pl.BlockSpec((NR // 2, BLK, H), lambda i: (0, i, 0)),
            pl.BlockSpec((BLK, H), lambda i: (i, 0)),
            pl.BlockSpec((NR, H, H), lambda i: (0, 0, 0)),
            pl.BlockSpec((H, H), lambda i: (0, 0)),
            pl.BlockSpec((1, H), lambda i: (0, 0)),
        ],
        out_specs=pl.BlockSpec((BLK, H), lambda i: (i, 0)),
        out_shape=jax.ShapeDtypeStruct((N, H), jnp.float32),
    )(Sa.reshape(NR // 2, N, H), Sb.reshape(NR // 2, N, H), x, W, wl,
      b.reshape(1, H))


def _z_body(sa_ref, sb_ref, h1_ref, wm_ref, ws_ref, wlm_ref, wls_ref,
            bm_ref, bs_ref, eps_ref, wcat_ref, z_ref, y_ref):
    zm = jnp.dot(h1_ref[...], wlm_ref[...], preferred_element_type=jnp.float32) + bm_ref[...]
    zs = jnp.dot(h1_ref[...], wls_ref[...], preferred_element_type=jnp.float32) + bs_ref[...]
    for r in range(NR // 2):
        zm = zm + jnp.dot(sa_ref[r], wm_ref[r], preferred_element_type=jnp.float32)
        zm = zm + jnp.dot(sb_ref[r], wm_ref[NR // 2 + r], preferred_element_type=jnp.float32)
        zs = zs + jnp.dot(sa_ref[r], ws_ref[r], preferred_element_type=jnp.float32)
        zs = zs + jnp.dot(sb_ref[r], ws_ref[NR // 2 + r], preferred_element_type=jnp.float32)
    z = zm + zs * eps_ref[...]
    z_ref[...] = z
    y_ref[...] = jnp.dot(z, wcat_ref[...], preferred_element_type=jnp.float32)


def _z_kernel(S, h1, Wm, Ws, wlm, wls, bm, bs, eps, Wcat):
    return pl.pallas_call(
        _z_body,
        grid=(_GRID,),
        in_specs=[
            pl.BlockSpec((NR // 2, BLK, H), lambda i: (0, i, 0)),
            pl.BlockSpec((NR // 2, BLK, H), lambda i: (0, i, 0)),
            pl.BlockSpec((BLK, H), lambda i: (i, 0)),
            pl.BlockSpec((NR, H, H), lambda i: (0, 0, 0)),
            pl.BlockSpec((NR, H, H), lambda i: (0, 0, 0)),
            pl.BlockSpec((H, H), lambda i: (0, 0)),
            pl.BlockSpec((H, H), lambda i: (0, 0)),
            pl.BlockSpec((1, H), lambda i: (0, 0)),
            pl.BlockSpec((1, H), lambda i: (0, 0)),
            pl.BlockSpec((BLK, H), lambda i: (i, 0)),
            pl.BlockSpec((H, 3 * H), lambda i: (0, 0)),
        ],
        out_specs=[
            pl.BlockSpec((BLK, H), lambda i: (i, 0)),
            pl.BlockSpec((BLK, 3 * H), lambda i: (i, 0)),
        ],
        out_shape=[
            jax.ShapeDtypeStruct((N, H), jnp.float32),
            jax.ShapeDtypeStruct((N, 3 * H), jnp.float32),
        ],
    )(S[0].reshape(NR // 2, N, H), S[1].reshape(NR // 2, N, H), h1, Wm, Ws,
      wlm, wls, bm.reshape(1, H), bs.reshape(1, H), eps, Wcat)


def _ru_body(hb_ref, hf_ref, y_ref, ysum_ref, skip_ref, sc_ref, o_ref):
    A = jax.nn.sigmoid(
        lax.dot_general(hb_ref[...].astype(jnp.bfloat16),
                        hf_ref[...].astype(jnp.bfloat16), (((1,), (1,)), ((), ())),
                        preferred_element_type=jnp.float32))
    rs = jnp.sum(A, axis=1, keepdims=True) + 1e-8
    Rm = A / rs
    out = (sc_ref[0, 0] * jnp.dot(Rm.astype(jnp.bfloat16),
                                  y_ref[...].astype(jnp.bfloat16),
                                  preferred_element_type=jnp.float32)
           + sc_ref[0, 1] * ysum_ref[...] + skip_ref[...])
    o_ref[...] = jnp.maximum(out, 0.0)


def _refine_update(hsrc, y, ysum, skip, scale_shift):
    return pl.pallas_call(
        _ru_body,
        grid=(_GRID,),
        in_specs=[
            pl.BlockSpec((BLK, H), lambda i: (i, 0)),
            pl.BlockSpec((N, H), lambda i: (0, 0)),
            pl.BlockSpec((N, H), lambda i: (0, 0)),
            pl.BlockSpec((1, H), lambda i: (0, 0)),
            pl.BlockSpec((BLK, H), lambda i: (i, 0)),
            pl.BlockSpec(memory_space=pltpu.SMEM),
        ],
        out_specs=pl.BlockSpec((BLK, H), lambda i: (i, 0)),
        out_shape=jax.ShapeDtypeStruct((N, H), jnp.float32),
    )(hsrc, hsrc, y, ysum, skip, scale_shift)


def _rm_body(hb_ref, hf_ref, r_ref, ec_ref):
    gb = jnp.maximum(hb_ref[...], 0.0)
    gf = jnp.maximum(hf_ref[...], 0.0)
    A = jax.nn.sigmoid(
        lax.dot_general(gb.astype(jnp.bfloat16), gf.astype(jnp.bfloat16),
                        (((1,), (1,)), ((), ())),
                        preferred_element_type=jnp.float32))
    Rm = A / (jnp.sum(A, axis=1, keepdims=True) + 1e-8)
    r_ref[...] = Rm
    ec_ref[...] = jnp.concatenate(
        [Rm[:, :1], Rm[:, N - 1:N], jnp.zeros((BLK, 126), jnp.float32)], axis=1)


def _refine_mat(hsrc):
    return pl.pallas_call(
        _rm_body,
        grid=(_GRID,),
        in_specs=[
            pl.BlockSpec((BLK, H), lambda i: (i, 0)),
            pl.BlockSpec((N, H), lambda i: (0, 0)),
        ],
        out_specs=[
            pl.BlockSpec((BLK, N), lambda i: (i, 0)),
            pl.BlockSpec((BLK, H), lambda i: (i, 0)),
        ],
        out_shape=[
            jax.ShapeDtypeStruct((N, N), jnp.float32),
            jax.ShapeDtypeStruct((N, H), jnp.float32),
        ],
    )(hsrc, hsrc)


def _mv_body(r2_ref, yt_ref, corr_ref, o_ref):
    o_ref[...] = jnp.maximum(
        jnp.dot(r2_ref[...].astype(jnp.bfloat16),
                yt_ref[...].astype(jnp.bfloat16),
                preferred_element_type=jnp.float32)
        + corr_ref[...], 0.0)


def _stencil_mv(R2, yt, corr):
    return pl.pallas_call(
        _mv_body,
        grid=(_GRID,),
        in_specs=[
            pl.BlockSpec((BLK, N), lambda i: (i, 0)),
            pl.BlockSpec((N, H), lambda i: (0, 0)),
            pl.BlockSpec((BLK, H), lambda i: (i, 0)),
        ],
        out_specs=pl.BlockSpec((BLK, H), lambda i: (i, 0)),
        out_shape=jax.ShapeDtypeStruct((N, H), jnp.float32),
    )(R2, yt, corr)


_NC, _NS = 2, 16
_EPT = E // _NS
_GSZ = 128
_NCHUNK = _EPT // _GSZ
_WROW = _EPT // 8
_ACC_ROWS = 2 * N
_RPT = _ACC_ROWS // _NS


def _scale_chunk(rows, wv, i, half):
    def _edge16(g16, c2):
        for j in range(2):
            wrow = i * (_GSZ // 8) + half * (_GSZ // 16) + g16 * 2 + j
            for k in range(8):
                wg = wv[wrow, pl.ds(k * 16, 16)]
                g = half * (_GSZ // 2) + g16 * 16 + j * 8 + k
                for f in range(H // 16):
                    rows[g, pl.ds(f * 16, 16)] = rows[g, pl.ds(f * 16, 16)] * wg
        return c2
    lax.fori_loop(0, _GSZ // 32, _edge16, 0)


def _sc_scatter_body(x_hbm, src_hbm, sidx_hbm, w_hbm, out_hbm,
                     srcv, sidxv, wv, rows0, rows1, stage, Sacc,
                     gsem0, gsem1, ssem0, ssem1):
    c = lax.axis_index("c")
    s = lax.axis_index("s")
    rows = (rows0, rows1)
    gsem = (gsem0, gsem1)
    ssem = (ssem0, ssem1)

    pltpu.sync_copy(src_hbm.at[s], srcv)
    pltpu.sync_copy(sidx_hbm.at[s], sidxv)
    pltpu.sync_copy(w_hbm.at[c, s], wv)

    def _zrow(i, carry):
        for f in range(H // 16):
            stage[i, pl.ds(f * 16, 16)] = jnp.zeros((16,), jnp.float32)
        return carry
    lax.fori_loop(0, 128, _zrow, 0)

    def _zcp(j, carry):
        pltpu.sync_copy(stage, Sacc.at[pl.ds(s * _RPT + j * 128, 128)])
        return carry
    lax.fori_loop(0, _RPT // 128, _zcp, 0)
    plsc.subcore_barrier()

    pltpu.async_copy(x_hbm.at[srcv.at[0]], rows0, gsem0)

    def _pair(i2, carry):
        for b in range(2):
            ch = i2 * 2 + b
            @pl.when(jnp.logical_and(ch >= 1, ch + 1 < _NCHUNK))
            def _():
                pltpu.make_async_copy(
                    rows[1 - b], Sacc.at[sidxv.at[ch - 1]], ssem[1 - b]).wait()

            @pl.when(ch + 1 < _NCHUNK)
            def _():
                pltpu.async_copy(x_hbm.at[srcv.at[ch + 1]], rows[1 - b], gsem[1 - b])

            pltpu.make_async_copy(x_hbm.at[srcv.at[ch]], rows[b], gsem[b]).wait()
            for half in range(2):
                _scale_chunk(rows[b], wv, ch, half)
            pltpu.async_copy(rows[b], Sacc.at[sidxv.at[ch]], ssem[b])
        return carry
    lax.fori_loop(0, _NCHUNK // 2, _pair, 0)

    pltpu.make_async_copy(rows0, Sacc.at[sidxv.at[_NCHUNK - 2]], ssem0).wait()
    pltpu.make_async_copy(rows1, Sacc.at[sidxv.at[_NCHUNK - 1]], ssem1).wait()
    plsc.subcore_barrier()

    off = s * _RPT
    pltpu.sync_copy(Sacc.at[pl.ds(off, _RPT)],
                    out_hbm.at[pl.ds(c * _ACC_ROWS + off, _RPT)])


_sc_scatter_call = functools.partial(
    pl.kernel,
    mesh=plsc.VectorSubcoreMesh(core_axis_name="c", subcore_axis_name="s"),
    out_type=jax.ShapeDtypeStruct((2 * _ACC_ROWS, H), jnp.float32),
    scratch_types=[
        pltpu.VMEM((_NCHUNK, _GSZ), jnp.int32),
        pltpu.VMEM((_NCHUNK, _GSZ), jnp.int32),
        pltpu.VMEM((_WROW, 128), jnp.float32),
        pltpu.VMEM((_GSZ, H), jnp.float32),
        pltpu.VMEM((_GSZ, H), jnp.float32),
        pltpu.VMEM((128, H), jnp.float32),
        pltpu.VMEM_SHARED((_ACC_ROWS, H), jnp.float32),
        pltpu.SemaphoreType.DMA,
        pltpu.SemaphoreType.DMA,
        pltpu.SemaphoreType.DMA,
        pltpu.SemaphoreType.DMA,
    ],
)

_sc_scatter_fn = _sc_scatter_call(_sc_scatter_body)


def _segment_scatter(x, src, sidx, wp):
    def one(p):
        wexp = jnp.broadcast_to(wp[p].reshape(_NC, _NS, _WROW, 8, 1),
                                (_NC, _NS, _WROW, 8, 16)).reshape(_NC, _NS, _WROW, 128)
        return _sc_scatter_fn(x,
                              src.reshape(_NS, _NCHUNK, _GSZ),
                              sidx.reshape(_NS, _NCHUNK, _GSZ),
                              wexp)
    return one(0), one(1)


def kernel(edge_index, h, r, norm, emb, rc_bases, rc_coef, rc_wloop, rc_bias,
           zm_bases, zm_coef, zm_wloop, zm_bias, zs_bases, zs_coef, zs_wloop,
           zs_bias, W1, W2, W3, conv1_w, conv1_b, conv2_w, conv2_b):
    src = edge_index[0].astype(jnp.int32)
    dst = edge_index[1].astype(jnp.int32)
    ri = r.astype(jnp.int32)
    x0 = emb

    sidx = (ri % 2) * N + dst
    nrm = norm[:, 0]
    rq = ri >> 1
    wp = jnp.stack([jnp.stack([jnp.where(rq == 2 * p + c, nrm, 0.0)
                               for c in range(_NC)]) for p in range(2)])

    Wrc = jnp.einsum('rb,bio->rio', rc_coef, rc_bases)
    Wzm = jnp.einsum('rb,bio->rio', zm_coef, zm_bases)
    Wzs = jnp.einsum('rb,bio->rio', zs_coef, zs_bases)
    Wcat = jnp.concatenate([W1, W2, W3], axis=1)

    eps = jax.random.normal(jax.random.key(1), (N, H), dtype=jnp.float32)

    S1 = _segment_scatter(x0, src, sidx, wp)
    h1 = _enc_post(S1, x0, Wrc, rc_wloop, rc_bias, act=True)
    S2 = _segment_scatter(h1, src, sidx, wp)
    z, y = _z_kernel(S2, h1, Wzm, Wzs, zm_wloop, zs_wloop, zm_bias, zs_bias,
                     eps, Wcat)
    y1, y2, y3 = y[:, :H], y[:, H:2 * H], y[:, 2 * H:]

    one_zero = jnp.array([[1.0, 0.0]], jnp.float32)
    hd1 = _refine_update(z, y1, jnp.zeros((1, H), jnp.float32), x0, one_zero)

    a = jnp.mean(conv1_w[:, 0])
    c = jnp.mean(conv1_b)
    ysum2 = jnp.sum(y2, axis=0, keepdims=True)
    hd2 = _refine_update(hd1, y2, ysum2, h1,
                         jnp.stack([a, c]).reshape(1, 2))

    R2, ec = _refine_mat(hd2)
    u = jnp.mean(jnp.sum(conv2_w, axis=1), axis=0)
    c2 = jnp.mean(conv2_b)
    y3p = jnp.concatenate([y3[1:], jnp.zeros((1, H), jnp.float32)], 0)
    y3m = jnp.concatenate([jnp.zeros((1, H), jnp.float32), y3[:-1]], 0)
    yt = u[1] * y3 + u[0] * y3p + u[2] * y3m
    firstcol, lastcol = ec[:, 0], ec[:, 1]
    prev_last = jnp.concatenate([jnp.zeros((1,), jnp.float32), lastcol[:-1]])
    next_first = jnp.concatenate([firstcol[1:], jnp.zeros((1,), jnp.float32)])
    corr = (u[0] * prev_last[:, None] * y3[0][None, :]
            + u[2] * next_first[:, None] * y3[N - 1][None, :]
            + c2 * jnp.sum(y3, axis=0)[None, :])
    hd3 = _stencil_mv(R2, yt, corr)

    out, _ = _refine_mat(hd3)
    return out
